# initial kernel scaffold (unmeasured)
import jax
import jax.numpy as jnp
from jax import lax
from jax.experimental import pallas as pl
from jax.experimental.pallas import tpu as pltpu

N_DEV = 4
N_EXPERTS = 32
N_STEPS = N_DEV - 1


def kernel(x, router_W, route_idx, expert_W):
    n_tok, d = x.shape
    e_loc, _, h = expert_W.shape
    chunk = n_tok // N_DEV

    def body(x_ref, rw_ref, idx_ref, ew_ref, out_ref,
             acc_ref, send_ref, recv_ref, send_sems, recv_sems):
        my = lax.axis_index("i")
        left = lax.rem(my + N_DEV - 1, N_DEV)
        right = lax.rem(my + 1, N_DEV)

        barrier_sem = pltpu.get_barrier_semaphore()
        for nbr in [left, right]:
            pl.semaphore_signal(
                barrier_sem, inc=1,
                device_id=(nbr,), device_id_type=pl.DeviceIdType.MESH,
            )
        pl.semaphore_wait(barrier_sem, 2)

        xf = x_ref[:, :]
        scores = jnp.dot(xf, rw_ref[:, :], preferred_element_type=jnp.float32)
        s_max = jnp.max(scores, axis=-1, keepdims=True)
        probs = jnp.exp(scores - s_max)
        probs = probs / jnp.sum(probs, axis=-1, keepdims=True)

        e0 = idx_ref[:, 0:1]
        e1 = idx_ref[:, 1:2]
        ids = lax.broadcasted_iota(jnp.int32, (n_tok, N_EXPERTS), 1)
        g0 = jnp.sum(jnp.where(ids == e0, probs, 0.0), axis=-1, keepdims=True)
        g1 = jnp.sum(jnp.where(ids == e1, probs, 0.0), axis=-1, keepdims=True)
        gs = g0 + g1
        g0 = g0 / gs
        g1 = g1 / gs

        base = my * e_loc
        acc = jnp.zeros((n_tok, h), jnp.float32)
        for le in range(e_loc):
            gid = base + le
            w = jnp.where(e0 == gid, g0, 0.0) + jnp.where(e1 == gid, g1, 0.0)
            xw = (xf * w).astype(jnp.bfloat16)
            acc = acc + jnp.dot(
                xw, ew_ref[le].astype(jnp.bfloat16),
                preferred_element_type=jnp.float32,
            )
        acc_ref[:, :] = acc

        for s in range(N_STEPS):
            send_chunk = lax.rem(my + N_DEV - 1 - s, N_DEV)
            data = acc_ref[pl.ds(send_chunk * chunk, chunk), :]
            if s > 0:
                data = data + recv_ref[s - 1]
            send_ref[s] = data
            rdma = pltpu.make_async_remote_copy(
                src_ref=send_ref.at[s],
                dst_ref=recv_ref.at[s],
                send_sem=send_sems.at[s],
                recv_sem=recv_sems.at[s],
                device_id=(right,),
                device_id_type=pl.DeviceIdType.MESH,
            )
            rdma.start()
            rdma.wait()

        out_ref[:, :] = (
            acc_ref[pl.ds(my * chunk, chunk), :] + recv_ref[N_STEPS - 1]
        )

    return pl.pallas_call(
        body,
        out_shape=jax.ShapeDtypeStruct((chunk, h), jnp.float32),
        in_specs=[pl.BlockSpec(memory_space=pltpu.VMEM)] * 4,
        out_specs=pl.BlockSpec(memory_space=pltpu.VMEM),
        scratch_shapes=[
            pltpu.VMEM((n_tok, h), jnp.float32),
            pltpu.VMEM((N_STEPS, chunk, h), jnp.float32),
            pltpu.VMEM((N_STEPS, chunk, h), jnp.float32),
            pltpu.SemaphoreType.DMA((N_STEPS,)),
            pltpu.SemaphoreType.DMA((N_STEPS,)),
        ],
        compiler_params=pltpu.CompilerParams(collective_id=0),
    )(x, router_W, route_idx, expert_W)


# baseline (device time: 97512 ns/iter reference)
import jax
import jax.numpy as jnp
from jax import lax
from jax.experimental import pallas as pl
from jax.experimental.pallas import tpu as pltpu

N_DEV = 4
N_EXPERTS = 32
N_STEPS = N_DEV - 1


def kernel(x, router_W, route_idx, expert_W):
    n_tok, d = x.shape
    e_loc, _, h = expert_W.shape
    chunk = n_tok // N_DEV

    def body(x_ref, rw_ref, idx_ref, ew_ref, out_ref,
             acc_ref, send_ref, recv_ref, send_sems, recv_sems):
        my = lax.axis_index("i")
        left = lax.rem(my + N_DEV - 1, N_DEV)
        right = lax.rem(my + 1, N_DEV)

        barrier_sem = pltpu.get_barrier_semaphore()
        for nbr in [left, right]:
            pl.semaphore_signal(
                barrier_sem, inc=1,
                device_id=(nbr,), device_id_type=pl.DeviceIdType.MESH,
            )
        pl.semaphore_wait(barrier_sem, 2)

        xb = x_ref[:, :]
        scores = jnp.dot(xb, rw_ref[:, :], preferred_element_type=jnp.float32)
        s_max = jnp.max(scores, axis=-1, keepdims=True)
        probs = jnp.exp(scores - s_max)
        probs = probs / jnp.sum(probs, axis=-1, keepdims=True)

        e0 = idx_ref[:, 0:1]
        e1 = idx_ref[:, 1:2]
        ids = lax.broadcasted_iota(jnp.int32, (n_tok, N_EXPERTS), 1)
        g0 = jnp.sum(jnp.where(ids == e0, probs, 0.0), axis=-1, keepdims=True)
        g1 = jnp.sum(jnp.where(ids == e1, probs, 0.0), axis=-1, keepdims=True)
        gs = g0 + g1
        g0 = g0 / gs
        g1 = g1 / gs

        base = my * e_loc
        for le in range(e_loc):
            gid = base + le
            w = jnp.where(e0 == gid, g0, 0.0) + jnp.where(e1 == gid, g1, 0.0)
            xw = (xb.astype(jnp.float32) * w).astype(jnp.bfloat16)
            part = jnp.dot(xw, ew_ref[le], preferred_element_type=jnp.float32)
            if le == 0:
                acc_ref[:, :] = part
            else:
                acc_ref[:, :] = acc_ref[:, :] + part

        for s in range(N_STEPS):
            send_chunk = lax.rem(my + N_DEV - 1 - s, N_DEV)
            data = acc_ref[pl.ds(send_chunk * chunk, chunk), :]
            if s > 0:
                data = data + recv_ref[s - 1].astype(jnp.float32)
            send_ref[s] = data.astype(jnp.bfloat16)
            rdma = pltpu.make_async_remote_copy(
                src_ref=send_ref.at[s],
                dst_ref=recv_ref.at[s],
                send_sem=send_sems.at[s],
                recv_sem=recv_sems.at[s],
                device_id=(right,),
                device_id_type=pl.DeviceIdType.MESH,
            )
            rdma.start()
            rdma.wait()

        out_ref[:, :] = (
            acc_ref[pl.ds(my * chunk, chunk), :]
            + recv_ref[N_STEPS - 1].astype(jnp.float32)
        )

    f = pl.pallas_call(
        body,
        out_shape=jax.ShapeDtypeStruct((chunk, h), jnp.float32),
        in_specs=[pl.BlockSpec(memory_space=pltpu.VMEM)] * 4,
        out_specs=pl.BlockSpec(memory_space=pltpu.VMEM),
        scratch_shapes=[
            pltpu.VMEM((n_tok, h), jnp.float32),
            pltpu.VMEM((N_STEPS, chunk, h), jnp.bfloat16),
            pltpu.VMEM((N_STEPS, chunk, h), jnp.bfloat16),
            pltpu.SemaphoreType.DMA((N_STEPS,)),
            pltpu.SemaphoreType.DMA((N_STEPS,)),
        ],
        compiler_params=pltpu.CompilerParams(
            collective_id=0,
            vmem_limit_bytes=100 * 1024 * 1024,
        ),
    )
    return f(
        x.astype(jnp.bfloat16),
        router_W.astype(jnp.bfloat16),
        route_idx,
        expert_W.astype(jnp.bfloat16),
    )


# device time: 75377 ns/iter; 1.2937x vs baseline; 1.2937x over previous
import jax
import jax.numpy as jnp
from jax import lax
from jax.experimental import pallas as pl
from jax.experimental.pallas import tpu as pltpu

N_DEV = 4
N_EXPERTS = 32
N_STEPS = N_DEV - 1


def kernel(x, router_W, route_idx, expert_W):
    n_tok, d = x.shape
    e_loc, _, h = expert_W.shape
    chunk = n_tok // N_DEV

    def body(x_ref, rw_ref, idx_ref, ew_ref, out_ref,
             send_ref, recv_ref, send_sems, recv_sems):
        my = lax.axis_index("i")
        left = lax.rem(my + N_DEV - 1, N_DEV)
        right = lax.rem(my + 1, N_DEV)

        barrier_sem = pltpu.get_barrier_semaphore()
        for nbr in [left, right]:
            pl.semaphore_signal(
                barrier_sem, inc=1,
                device_id=(nbr,), device_id_type=pl.DeviceIdType.MESH,
            )
        pl.semaphore_wait(barrier_sem, 2)

        base = my * e_loc
        rdmas = []
        for s in range(N_DEV):
            c = lax.rem(my + N_DEV - 1 - s, N_DEV)
            rows = pl.ds(c * chunk, chunk)
            xc = x_ref[rows, :]

            scores = jnp.dot(
                xc, rw_ref[:, :], preferred_element_type=jnp.float32
            )
            s_max = jnp.max(scores, axis=-1, keepdims=True)
            probs = jnp.exp(scores - s_max)
            probs = probs / jnp.sum(probs, axis=-1, keepdims=True)
            e0 = idx_ref[rows, 0:1]
            e1 = idx_ref[rows, 1:2]
            ids = lax.broadcasted_iota(jnp.int32, (chunk, N_EXPERTS), 1)
            g0 = jnp.sum(
                jnp.where(ids == e0, probs, 0.0), axis=-1, keepdims=True
            )
            g1 = jnp.sum(
                jnp.where(ids == e1, probs, 0.0), axis=-1, keepdims=True
            )
            gs = g0 + g1
            g0 = g0 / gs
            g1 = g1 / gs

            xf = xc.astype(jnp.float32)
            part = jnp.zeros((chunk, h), jnp.float32)
            for le in range(e_loc):
                gid = base + le
                w = jnp.where(e0 == gid, g0, 0.0) + jnp.where(e1 == gid, g1, 0.0)
                xw = (xf * w).astype(jnp.bfloat16)
                part = part + jnp.dot(
                    xw, ew_ref[le], preferred_element_type=jnp.float32
                )

            if s > 0:
                rdmas[s - 1].wait_recv()
                part = part + recv_ref[s - 1].astype(jnp.float32)
            if s < N_STEPS:
                send_ref[s] = part.astype(jnp.bfloat16)
                rdma = pltpu.make_async_remote_copy(
                    src_ref=send_ref.at[s],
                    dst_ref=recv_ref.at[s],
                    send_sem=send_sems.at[s],
                    recv_sem=recv_sems.at[s],
                    device_id=(right,),
                    device_id_type=pl.DeviceIdType.MESH,
                )
                rdma.start()
                rdmas.append(rdma)
            else:
                out_ref[:, :] = part

        for rdma in rdmas:
            rdma.wait_send()

    f = pl.pallas_call(
        body,
        out_shape=jax.ShapeDtypeStruct((chunk, h), jnp.float32),
        in_specs=[pl.BlockSpec(memory_space=pltpu.VMEM)] * 4,
        out_specs=pl.BlockSpec(memory_space=pltpu.VMEM),
        scratch_shapes=[
            pltpu.VMEM((N_STEPS, chunk, h), jnp.bfloat16),
            pltpu.VMEM((N_STEPS, chunk, h), jnp.bfloat16),
            pltpu.SemaphoreType.DMA((N_STEPS,)),
            pltpu.SemaphoreType.DMA((N_STEPS,)),
        ],
        compiler_params=pltpu.CompilerParams(
            collective_id=0,
            vmem_limit_bytes=100 * 1024 * 1024,
        ),
    )
    return f(
        x.astype(jnp.bfloat16),
        router_W.astype(jnp.bfloat16),
        route_idx,
        expert_W.astype(jnp.bfloat16),
    )


# device time: 65887 ns/iter; 1.4800x vs baseline; 1.1440x over previous
import jax
import jax.numpy as jnp
from jax import lax
from jax.experimental import pallas as pl
from jax.experimental.pallas import tpu as pltpu

N_DEV = 4
N_EXPERTS = 32
N_STEPS = N_DEV - 1


def kernel(x, router_W, route_idx, expert_W):
    n_tok, d = x.shape
    e_loc, _, h = expert_W.shape
    chunk = n_tok // N_DEV

    def body(x_ref, rw_ref, idx_ref, ew_ref, out_ref,
             send_ref, recv_ref, send_sems, recv_sems):
        my = lax.axis_index("i")
        left = lax.rem(my + N_DEV - 1, N_DEV)
        right = lax.rem(my + 1, N_DEV)

        barrier_sem = pltpu.get_barrier_semaphore()
        for nbr in [left, right]:
            pl.semaphore_signal(
                barrier_sem, inc=1,
                device_id=(nbr,), device_id_type=pl.DeviceIdType.MESH,
            )
        pl.semaphore_wait(barrier_sem, 2)

        base = my * e_loc
        rdmas = []
        for s in range(N_DEV):
            c = lax.rem(my + N_DEV - 1 - s, N_DEV)
            rows = pl.ds(c * chunk, chunk)
            xc = x_ref[rows, :]

            scores = jnp.dot(
                xc, rw_ref[:, :], preferred_element_type=jnp.float32
            )
            s_max = jnp.max(scores, axis=-1, keepdims=True)
            probs = jnp.exp(scores - s_max)
            probs = probs / jnp.sum(probs, axis=-1, keepdims=True)
            e0 = idx_ref[rows, 0:1]
            e1 = idx_ref[rows, 1:2]
            ids = lax.broadcasted_iota(jnp.int32, (chunk, N_EXPERTS), 1)
            g0 = jnp.sum(
                jnp.where(ids == e0, probs, 0.0), axis=-1, keepdims=True
            )
            g1 = jnp.sum(
                jnp.where(ids == e1, probs, 0.0), axis=-1, keepdims=True
            )
            gs = g0 + g1
            g0 = g0 / gs
            g1 = g1 / gs

            part = jnp.zeros((chunk, h), jnp.float32)
            for le in range(e_loc):
                gid = base + le
                w = jnp.where(e0 == gid, g0, 0.0) + jnp.where(e1 == gid, g1, 0.0)
                part = part + jnp.dot(
                    xc * w, ew_ref[le],
                    preferred_element_type=jnp.float32,
                    precision=lax.Precision.DEFAULT,
                )

            if s > 0:
                rdmas[s - 1].wait_recv()
                part = part + recv_ref[s - 1].astype(jnp.float32)
            if s < N_STEPS:
                send_ref[s] = part.astype(jnp.bfloat16)
                rdma = pltpu.make_async_remote_copy(
                    src_ref=send_ref.at[s],
                    dst_ref=recv_ref.at[s],
                    send_sem=send_sems.at[s],
                    recv_sem=recv_sems.at[s],
                    device_id=(right,),
                    device_id_type=pl.DeviceIdType.MESH,
                )
                rdma.start()
                rdmas.append(rdma)
            else:
                out_ref[:, :] = part

        for rdma in rdmas:
            rdma.wait_send()

    f = pl.pallas_call(
        body,
        out_shape=jax.ShapeDtypeStruct((chunk, h), jnp.float32),
        in_specs=[pl.BlockSpec(memory_space=pltpu.VMEM)] * 4,
        out_specs=pl.BlockSpec(memory_space=pltpu.VMEM),
        scratch_shapes=[
            pltpu.VMEM((N_STEPS, chunk, h), jnp.bfloat16),
            pltpu.VMEM((N_STEPS, chunk, h), jnp.bfloat16),
            pltpu.SemaphoreType.DMA((N_STEPS,)),
            pltpu.SemaphoreType.DMA((N_STEPS,)),
        ],
        compiler_params=pltpu.CompilerParams(
            collective_id=0,
            vmem_limit_bytes=100 * 1024 * 1024,
        ),
    )
    return f(x, router_W, route_idx, expert_W)
